# baseline (speedup 1.0000x reference)
import functools

import jax
import jax.numpy as jnp
from jax.experimental import pallas as pl

MASK_PROP = 0.75


def _mask_encode_kernel(num_mask, p_ref, rl_ref, rs_ref, m_ref, e_ref, i_ref):
    n = p_ref.shape[1]
    num_unmask = n - num_mask
    chunk = 48

    v = rl_ref[0, 0, :][None, :]
    rank = jnp.zeros((1, n), jnp.float32)
    for c in range(0, n, chunk):
        vj = rs_ref[0, c : c + chunk, :]
        rank = rank + jnp.sum((vj < v).astype(jnp.float32), axis=0, keepdims=True)
    ranki = rank.astype(jnp.int32)

    kk = jax.lax.broadcasted_iota(jnp.int32, (num_unmask, n), 0) + num_mask
    onehot = (ranki == kk).astype(jnp.float32)
    e_ref[0, :num_unmask, :] = jnp.dot(
        onehot, p_ref[0], preferred_element_type=jnp.float32
    )
    e_ref[0, num_unmask:, :] = jnp.broadcast_to(
        m_ref[0, :], (num_mask, e_ref.shape[2])
    )

    mk = jax.lax.broadcasted_iota(jnp.int32, (num_mask, n), 0)
    msel = (ranki == mk).astype(jnp.float32)
    col = jax.lax.broadcasted_iota(jnp.int32, (n, 1), 0).astype(jnp.float32)
    idxf = jnp.dot(msel, col, precision=jax.lax.Precision.HIGHEST)
    i_ref[0, :, :] = idxf.astype(jnp.int32)


def kernel(patches, mask_token):
    b, n, e = patches.shape
    num_mask = -(-3 * n // 4)

    rkey = jax.random.key(42)
    rand_vals = jax.random.uniform(rkey, (b, n), dtype=jnp.float32)
    rand_lane = rand_vals.reshape(b, 1, n)
    rand_sub = rand_vals.reshape(b, n, 1)

    enc, idx3 = pl.pallas_call(
        functools.partial(_mask_encode_kernel, num_mask),
        grid=(b,),
        in_specs=[
            pl.BlockSpec((1, n, e), lambda i: (i, 0, 0)),
            pl.BlockSpec((1, 1, n), lambda i: (i, 0, 0)),
            pl.BlockSpec((1, n, 1), lambda i: (i, 0, 0)),
            pl.BlockSpec((1, e), lambda i: (0, 0)),
        ],
        out_specs=[
            pl.BlockSpec((1, n, e), lambda i: (i, 0, 0)),
            pl.BlockSpec((1, num_mask, 1), lambda i: (i, 0, 0)),
        ],
        out_shape=[
            jax.ShapeDtypeStruct((b, n, e), jnp.float32),
            jax.ShapeDtypeStruct((b, num_mask, 1), jnp.int32),
        ],
    )(patches, rand_lane, rand_sub, mask_token)
    return enc, idx3.reshape(b, num_mask)
